# initial kernel scaffold (unmeasured)
import jax
import jax.numpy as jnp
from jax import lax
from jax.experimental import pallas as pl
from jax.experimental.pallas import tpu as pltpu


def kernel(
    x,
):
    def body(*refs):
        pass

    out_shape = jax.ShapeDtypeStruct(..., jnp.float32)
    return pl.pallas_call(body, out_shape=out_shape)(...)


# baseline (device time: 17763 ns/iter reference)
import jax
import jax.numpy as jnp
from jax import lax
from jax.experimental import pallas as pl
from jax.experimental.pallas import tpu as pltpu

N_DEV = 32


def kernel(x):
    m_per, n = x.shape

    def body(x_ref, out_ref, carry_ref, send_buf_ref, send_sem, recv_sem):
        my = lax.axis_index("i")

        a = x_ref[:, :]
        k = 1
        while k < m_per:
            shifted = jnp.concatenate(
                [jnp.ones((k, n), a.dtype), a[: m_per - k, :]], axis=0
            )
            a = a * shifted
            k *= 2
        total = a[m_per - 1 : m_per, :]

        @pl.when(my == 0)
        def _():
            carry_ref[:, :] = jnp.ones((1, n), jnp.float32)

        @pl.when(my > 0)
        def _():
            recv = pltpu.make_async_remote_copy(
                src_ref=send_buf_ref,
                dst_ref=carry_ref,
                send_sem=send_sem,
                recv_sem=recv_sem,
                device_id=(my - 1,),
                device_id_type=pl.DeviceIdType.MESH,
            )
            recv.wait_recv()

        @pl.when(my < N_DEV - 1)
        def _():
            send_buf_ref[:, :] = carry_ref[:, :] * total
            send = pltpu.make_async_remote_copy(
                src_ref=send_buf_ref,
                dst_ref=carry_ref,
                send_sem=send_sem,
                recv_sem=recv_sem,
                device_id=(my + 1,),
                device_id_type=pl.DeviceIdType.MESH,
            )
            send.start()
            send.wait_send()

        out_ref[:, :] = a * carry_ref[:, :]

    return pl.pallas_call(
        body,
        out_shape=jax.ShapeDtypeStruct((m_per, n), jnp.float32),
        in_specs=[pl.BlockSpec(memory_space=pltpu.VMEM)],
        out_specs=pl.BlockSpec(memory_space=pltpu.VMEM),
        scratch_shapes=[
            pltpu.VMEM((1, n), jnp.float32),
            pltpu.VMEM((1, n), jnp.float32),
            pltpu.SemaphoreType.DMA,
            pltpu.SemaphoreType.DMA,
        ],
    )(x)
